# baseline (speedup 1.0000x reference)
import functools

import jax
import jax.numpy as jnp
from jax import lax
from jax.experimental import pallas as pl
from jax.experimental.pallas import tpu as pltpu
from jax.experimental.pallas import tpu_sc as plsc

HB = 64
NBINS = 32768
NW = 32
BUF = 8192


def _ce_block(x, t):
    m = jnp.max(x, axis=0)
    e = jnp.exp(x - m[None, :, :])
    se = jnp.sum(e, axis=0)
    lse = m + jnp.log(se)
    cls = lax.broadcasted_iota(jnp.int32, x.shape, 0)
    xt = jnp.sum(jnp.where(cls == t[None, :, :], x, 0.0), axis=0)
    ce = jnp.maximum(lse - xt, 0.0)
    return ce, xt, m, e, se


def _scal_body(preds_ref, tgt_ref, scal_ref, *, ntot):
    b = pl.program_id(0)
    h = pl.program_id(1)
    x = preds_ref[0]
    t = tgt_ref[0]
    ce, xt, m, e, _ = _ce_block(x, t)
    a = jnp.exp(jnp.minimum(m, 80.0))
    t1 = e * a[None, :, :]
    s_part = jnp.sum(t1 / (1.0 + t1))
    i_part = jnp.sum(1.0 / (1.0 + jnp.exp(-xt)))
    thresh = -jnp.log(jnp.float32(0.7))
    hard = ce > thresh
    hc_part = jnp.sum(hard.astype(jnp.float32))
    hs_part = jnp.sum(jnp.where(hard, ce, 0.0))

    @pl.when((b == 0) & (h == 0))
    def _init():
        scal_ref[...] = jnp.zeros_like(scal_ref)

    row = lambda v: jnp.full((1, 128), v, jnp.float32)
    upd = jnp.concatenate(
        [row(s_part), row(i_part), row(hc_part), row(hs_part),
         jnp.zeros((4, 128), jnp.float32)], axis=0)
    scal_ref[...] += upd

    @pl.when((b == pl.num_programs(0) - 1) & (h == pl.num_programs(1) - 1))
    def _final():
        iouloss, mean_hard, _ = _iou_and_hard(scal_ref, ntot)
        scal_ref[4:5, :] = jnp.full((1, 128), 2.0 * iouloss + mean_hard,
                                    jnp.float32)


def _run_scal(preds, targets):
    B, C, H, W = preds.shape
    return pl.pallas_call(
        functools.partial(_scal_body, ntot=float(targets.size)),
        grid=(B, H // HB),
        in_specs=[
            pl.BlockSpec((1, C, HB, W), lambda b, h: (b, 0, h, 0)),
            pl.BlockSpec((1, HB, W), lambda b, h: (b, h, 0)),
        ],
        out_specs=pl.BlockSpec((8, 128), lambda b, h: (0, 0)),
        out_shape=jax.ShapeDtypeStruct((8, 128), jnp.float32),
    )(preds, targets)


def _ce_body(preds_ref, tgt_ref, ce_ref, bn_ref):
    x = preds_ref[0]
    t = tgt_ref[0]
    ce, _, _, _, _ = _ce_block(x, t)
    ce_ref[0] = ce
    bn_ref[0] = lax.shift_right_logical(
        lax.bitcast_convert_type(ce, jnp.int32), 16)


def _run_ce(preds, targets):
    B, C, H, W = preds.shape
    return pl.pallas_call(
        _ce_body,
        grid=(B, H // HB),
        in_specs=[
            pl.BlockSpec((1, C, HB, W), lambda b, h: (b, 0, h, 0)),
            pl.BlockSpec((1, HB, W), lambda b, h: (b, h, 0)),
        ],
        out_specs=[
            pl.BlockSpec((1, HB, W), lambda b, h: (b, h, 0)),
            pl.BlockSpec((1, HB, W), lambda b, h: (b, h, 0)),
        ],
        out_shape=[
            jax.ShapeDtypeStruct((B, H, W), jnp.float32),
            jax.ShapeDtypeStruct((B, H, W), jnp.int32),
        ],
    )(preds, targets)


def _hist_body(ce_hbm, bn_hbm, zeros_hbm, cnt_hbm, sum_hbm,
               buf_v, bnbuf_v, hcnt_v, hsum_v):
    chunk = ce_hbm.shape[0] // NW
    c = lax.axis_index("c")
    s = lax.axis_index("s")
    wid = s * 2 + c
    base = wid * chunk
    pltpu.sync_copy(zeros_hbm, hcnt_v)
    pltpu.sync_copy(zeros_hbm, hsum_v)
    ones = jnp.full((16,), 1.0, jnp.float32)

    def outer(ib, carry):
        pltpu.sync_copy(ce_hbm.at[pl.ds(base + ib * BUF, BUF)], buf_v)
        pltpu.sync_copy(bn_hbm.at[pl.ds(base + ib * BUF, BUF)], bnbuf_v)

        def inner(j, carry2):
            for u in range(4):
                v = buf_v[pl.ds(j * 64 + u * 16, 16)]
                bn = bnbuf_v[pl.ds(j * 64 + u * 16, 16)]
                plsc.addupdate_scatter(hcnt_v, [bn], ones)
                plsc.addupdate_scatter(hsum_v, [bn], v)
            return carry2

        lax.fori_loop(0, BUF // 64, inner, 0)
        return carry

    lax.fori_loop(0, chunk // BUF, outer, 0)
    pltpu.sync_copy(hcnt_v, cnt_hbm.at[wid])
    pltpu.sync_copy(hsum_v, sum_hbm.at[wid])


def _run_hist(ce_flat, bn_flat):
    mesh = plsc.VectorSubcoreMesh(core_axis_name="c", subcore_axis_name="s")
    zeros = jnp.zeros((NBINS,), jnp.float32)
    fn = functools.partial(
        pl.kernel,
        mesh=mesh,
        compiler_params=pltpu.CompilerParams(needs_layout_passes=False),
        out_type=(
            jax.ShapeDtypeStruct((NW, NBINS), jnp.float32),
            jax.ShapeDtypeStruct((NW, NBINS), jnp.float32),
        ),
        scratch_types=[
            pltpu.VMEM((BUF,), jnp.float32),
            pltpu.VMEM((BUF,), jnp.int32),
            pltpu.VMEM((NBINS,), jnp.float32),
            pltpu.VMEM((NBINS,), jnp.float32),
        ],
    )(_hist_body)
    return fn(ce_flat, bn_flat, zeros)


def _flat_suffix(rows, w_mat, us_mat):
    rowsuf = jnp.dot(rows, w_mat, preferred_element_type=jnp.float32)
    r2 = jnp.dot(us_mat, rows, preferred_element_type=jnp.float32)
    tail = jnp.sum(r2, axis=1, keepdims=True)
    return rowsuf + tail


def _pick(onehot, vals):
    return jnp.sum(jnp.where(onehot, vals, 0.0))


def _iou_and_hard(scal_ref, ntot):
    s_all = scal_ref[0, 0]
    inter = scal_ref[1, 0]
    hc = scal_ref[2, 0]
    hs = scal_ref[3, 0]
    total = s_all + ntot
    union = total - inter
    iou = (inter + 1.0) / (union + 1.0)
    iouloss = 1.0 - iou
    mean_hard = hs / jnp.maximum(hc, 1.0)
    return iouloss, mean_hard, hc


def _select_body(cnt_ref, sum_ref, scal_ref, out_ref, *, kf, ntot):
    nr = NBINS // 128
    rows = jnp.sum(cnt_ref[...], axis=0).reshape(nr, 128)
    hrows = jnp.sum(sum_ref[...], axis=0).reshape(nr, 128)
    w_mat = (lax.broadcasted_iota(jnp.int32, (128, 128), 0)
             >= lax.broadcasted_iota(jnp.int32, (128, 128), 1)
             ).astype(jnp.float32)
    us_mat = (lax.broadcasted_iota(jnp.int32, (nr, nr), 1)
              > lax.broadcasted_iota(jnp.int32, (nr, nr), 0)
              ).astype(jnp.float32)
    f_cnt = _flat_suffix(rows, w_mat, us_mat)
    f_sum = _flat_suffix(hrows, w_mat, us_mat)
    nsel = jnp.sum((f_cnt >= kf).astype(jnp.int32))
    fid = (lax.broadcasted_iota(jnp.int32, (nr, 128), 0) * 128
           + lax.broadcasted_iota(jnp.int32, (nr, 128), 1))
    onehot = fid == (nsel - 1)
    bin_cnt = _pick(onehot, rows)
    bin_sum = _pick(onehot, hrows)
    cnt_gt = _pick(onehot, f_cnt) - bin_cnt
    sum_gt = _pick(onehot, f_sum) - bin_sum
    r = kf - cnt_gt
    topk_sum = sum_gt + r * (bin_sum / jnp.maximum(bin_cnt, 1.0))
    mean_topk = topk_sum / kf

    iouloss, mean_hard, hc = _iou_and_hard(scal_ref, ntot)
    ohem = jnp.where(hc < kf, mean_topk, mean_hard)
    out_ref[0, 0] = 2.0 * iouloss + ohem


def _run_select(cnt_h, sum_h, scal, kf, ntot):
    body = functools.partial(_select_body, kf=kf, ntot=ntot)
    return pl.pallas_call(
        body,
        in_specs=[
            pl.BlockSpec(memory_space=pltpu.VMEM),
            pl.BlockSpec(memory_space=pltpu.VMEM),
            pl.BlockSpec(memory_space=pltpu.VMEM),
        ],
        out_specs=pl.BlockSpec(memory_space=pltpu.SMEM),
        out_shape=jax.ShapeDtypeStruct((1, 1), jnp.float32),
    )(cnt_h, sum_h, scal)


def kernel(preds, targets):
    n_min = targets.size // 16
    kf = float(n_min)
    ntot = float(targets.size)
    scal = _run_scal(preds, targets)
    hc = scal[2, 0]

    def _hard_path(p, t, sc):
        return sc[4:5, 0:1]

    def _topk_path(p, t, sc):
        ce, bn = _run_ce(p, t)
        cnt_h, sum_h = _run_hist(ce.reshape(-1), bn.reshape(-1))
        return _run_select(cnt_h, sum_h, sc, kf, ntot)

    out = lax.cond(hc >= kf, _hard_path, _topk_path, preds, targets, scal)
    return out[0, 0]


# speedup vs baseline: 1.1810x; 1.1810x over previous
import functools

import jax
import jax.numpy as jnp
from jax import lax
from jax.experimental import pallas as pl
from jax.experimental.pallas import tpu as pltpu
from jax.experimental.pallas import tpu_sc as plsc

HB = 64
NBINS = 32768
NW = 32
BUF = 8192


def _ce_block(x, t):
    m = jnp.max(x, axis=0)
    e = jnp.exp(x - m[None, :, :])
    se = jnp.sum(e, axis=0)
    lse = m + jnp.log(se)
    cls = lax.broadcasted_iota(jnp.int32, x.shape, 0)
    xt = jnp.sum(jnp.where(cls == t[None, :, :], x, 0.0), axis=0)
    ce = jnp.maximum(lse - xt, 0.0)
    return ce, xt, m, e, se


def _scal_body(preds_ref, tgt_ref, scal_ref, *, ntot):
    b = pl.program_id(0)
    h = pl.program_id(1)
    C = preds_ref.shape[1]
    W = preds_ref.shape[3]
    m = jnp.max(preds_ref[0], axis=0)
    thresh = -jnp.log(jnp.float32(0.7))
    zero = jnp.zeros((8, W), jnp.float32)
    s_rcp = jnp.float32(0.0)
    i_part = jnp.float32(0.0)
    hc_part = jnp.float32(0.0)
    hs_part = jnp.float32(0.0)
    for r in range(HB // 8):
        rs = r * 8
        m_sub = lax.slice(m, (rs, 0), (rs + 8, W))
        a_sub = jnp.exp(jnp.minimum(m_sub, 80.0))
        t_sub = tgt_ref[0, rs:rs + 8, :]
        se = zero
        srcp = zero
        xt = zero
        for c in range(C):
            xc = preds_ref[0, c, rs:rs + 8, :]
            ec = jnp.exp(xc - m_sub)
            se = se + ec
            srcp = srcp + 1.0 / (1.0 + ec * a_sub)
            xt = xt + jnp.where(t_sub == c, xc, 0.0)
        lse = m_sub + jnp.log(se)
        ce = jnp.maximum(lse - xt, 0.0)
        hard = ce > thresh
        s_rcp = s_rcp + jnp.sum(srcp)
        i_part = i_part + jnp.sum(1.0 / (1.0 + jnp.exp(-xt)))
        hc_part = hc_part + jnp.sum(hard.astype(jnp.float32))
        hs_part = hs_part + jnp.sum(jnp.where(hard, ce, 0.0))
    s_part = jnp.float32(C * HB * W) - s_rcp

    @pl.when((b == 0) & (h == 0))
    def _init():
        scal_ref[...] = jnp.zeros_like(scal_ref)

    row = lambda v: jnp.full((1, 128), v, jnp.float32)
    upd = jnp.concatenate(
        [row(s_part), row(i_part), row(hc_part), row(hs_part),
         jnp.zeros((4, 128), jnp.float32)], axis=0)
    scal_ref[...] += upd

    @pl.when((b == pl.num_programs(0) - 1) & (h == pl.num_programs(1) - 1))
    def _final():
        iouloss, mean_hard, _ = _iou_and_hard(scal_ref, ntot)
        scal_ref[4:5, :] = jnp.full((1, 128), 2.0 * iouloss + mean_hard,
                                    jnp.float32)


def _run_scal(preds, targets):
    B, C, H, W = preds.shape
    return pl.pallas_call(
        functools.partial(_scal_body, ntot=float(targets.size)),
        grid=(B, H // HB),
        in_specs=[
            pl.BlockSpec((1, C, HB, W), lambda b, h: (b, 0, h, 0)),
            pl.BlockSpec((1, HB, W), lambda b, h: (b, h, 0)),
        ],
        out_specs=pl.BlockSpec((8, 128), lambda b, h: (0, 0)),
        out_shape=jax.ShapeDtypeStruct((8, 128), jnp.float32),
    )(preds, targets)


def _ce_body(preds_ref, tgt_ref, ce_ref, bn_ref):
    x = preds_ref[0]
    t = tgt_ref[0]
    ce, _, _, _, _ = _ce_block(x, t)
    ce_ref[0] = ce
    bn_ref[0] = lax.shift_right_logical(
        lax.bitcast_convert_type(ce, jnp.int32), 16)


def _run_ce(preds, targets):
    B, C, H, W = preds.shape
    return pl.pallas_call(
        _ce_body,
        grid=(B, H // HB),
        in_specs=[
            pl.BlockSpec((1, C, HB, W), lambda b, h: (b, 0, h, 0)),
            pl.BlockSpec((1, HB, W), lambda b, h: (b, h, 0)),
        ],
        out_specs=[
            pl.BlockSpec((1, HB, W), lambda b, h: (b, h, 0)),
            pl.BlockSpec((1, HB, W), lambda b, h: (b, h, 0)),
        ],
        out_shape=[
            jax.ShapeDtypeStruct((B, H, W), jnp.float32),
            jax.ShapeDtypeStruct((B, H, W), jnp.int32),
        ],
    )(preds, targets)


def _hist_body(ce_hbm, bn_hbm, zeros_hbm, cnt_hbm, sum_hbm,
               buf_v, bnbuf_v, hcnt_v, hsum_v):
    chunk = ce_hbm.shape[0] // NW
    c = lax.axis_index("c")
    s = lax.axis_index("s")
    wid = s * 2 + c
    base = wid * chunk
    pltpu.sync_copy(zeros_hbm, hcnt_v)
    pltpu.sync_copy(zeros_hbm, hsum_v)
    ones = jnp.full((16,), 1.0, jnp.float32)

    def outer(ib, carry):
        pltpu.sync_copy(ce_hbm.at[pl.ds(base + ib * BUF, BUF)], buf_v)
        pltpu.sync_copy(bn_hbm.at[pl.ds(base + ib * BUF, BUF)], bnbuf_v)

        def inner(j, carry2):
            for u in range(4):
                v = buf_v[pl.ds(j * 64 + u * 16, 16)]
                bn = bnbuf_v[pl.ds(j * 64 + u * 16, 16)]
                plsc.addupdate_scatter(hcnt_v, [bn], ones)
                plsc.addupdate_scatter(hsum_v, [bn], v)
            return carry2

        lax.fori_loop(0, BUF // 64, inner, 0)
        return carry

    lax.fori_loop(0, chunk // BUF, outer, 0)
    pltpu.sync_copy(hcnt_v, cnt_hbm.at[wid])
    pltpu.sync_copy(hsum_v, sum_hbm.at[wid])


def _run_hist(ce_flat, bn_flat):
    mesh = plsc.VectorSubcoreMesh(core_axis_name="c", subcore_axis_name="s")
    zeros = jnp.zeros((NBINS,), jnp.float32)
    fn = functools.partial(
        pl.kernel,
        mesh=mesh,
        compiler_params=pltpu.CompilerParams(needs_layout_passes=False),
        out_type=(
            jax.ShapeDtypeStruct((NW, NBINS), jnp.float32),
            jax.ShapeDtypeStruct((NW, NBINS), jnp.float32),
        ),
        scratch_types=[
            pltpu.VMEM((BUF,), jnp.float32),
            pltpu.VMEM((BUF,), jnp.int32),
            pltpu.VMEM((NBINS,), jnp.float32),
            pltpu.VMEM((NBINS,), jnp.float32),
        ],
    )(_hist_body)
    return fn(ce_flat, bn_flat, zeros)


def _flat_suffix(rows, w_mat, us_mat):
    rowsuf = jnp.dot(rows, w_mat, preferred_element_type=jnp.float32)
    r2 = jnp.dot(us_mat, rows, preferred_element_type=jnp.float32)
    tail = jnp.sum(r2, axis=1, keepdims=True)
    return rowsuf + tail


def _pick(onehot, vals):
    return jnp.sum(jnp.where(onehot, vals, 0.0))


def _iou_and_hard(scal_ref, ntot):
    s_all = scal_ref[0, 0]
    inter = scal_ref[1, 0]
    hc = scal_ref[2, 0]
    hs = scal_ref[3, 0]
    total = s_all + ntot
    union = total - inter
    iou = (inter + 1.0) / (union + 1.0)
    iouloss = 1.0 - iou
    mean_hard = hs / jnp.maximum(hc, 1.0)
    return iouloss, mean_hard, hc


def _select_body(cnt_ref, sum_ref, scal_ref, out_ref, *, kf, ntot):
    nr = NBINS // 128
    rows = jnp.sum(cnt_ref[...], axis=0).reshape(nr, 128)
    hrows = jnp.sum(sum_ref[...], axis=0).reshape(nr, 128)
    w_mat = (lax.broadcasted_iota(jnp.int32, (128, 128), 0)
             >= lax.broadcasted_iota(jnp.int32, (128, 128), 1)
             ).astype(jnp.float32)
    us_mat = (lax.broadcasted_iota(jnp.int32, (nr, nr), 1)
              > lax.broadcasted_iota(jnp.int32, (nr, nr), 0)
              ).astype(jnp.float32)
    f_cnt = _flat_suffix(rows, w_mat, us_mat)
    f_sum = _flat_suffix(hrows, w_mat, us_mat)
    nsel = jnp.sum((f_cnt >= kf).astype(jnp.int32))
    fid = (lax.broadcasted_iota(jnp.int32, (nr, 128), 0) * 128
           + lax.broadcasted_iota(jnp.int32, (nr, 128), 1))
    onehot = fid == (nsel - 1)
    bin_cnt = _pick(onehot, rows)
    bin_sum = _pick(onehot, hrows)
    cnt_gt = _pick(onehot, f_cnt) - bin_cnt
    sum_gt = _pick(onehot, f_sum) - bin_sum
    r = kf - cnt_gt
    topk_sum = sum_gt + r * (bin_sum / jnp.maximum(bin_cnt, 1.0))
    mean_topk = topk_sum / kf

    iouloss, mean_hard, hc = _iou_and_hard(scal_ref, ntot)
    ohem = jnp.where(hc < kf, mean_topk, mean_hard)
    out_ref[0, 0] = 2.0 * iouloss + ohem


def _run_select(cnt_h, sum_h, scal, kf, ntot):
    body = functools.partial(_select_body, kf=kf, ntot=ntot)
    return pl.pallas_call(
        body,
        in_specs=[
            pl.BlockSpec(memory_space=pltpu.VMEM),
            pl.BlockSpec(memory_space=pltpu.VMEM),
            pl.BlockSpec(memory_space=pltpu.VMEM),
        ],
        out_specs=pl.BlockSpec(memory_space=pltpu.SMEM),
        out_shape=jax.ShapeDtypeStruct((1, 1), jnp.float32),
    )(cnt_h, sum_h, scal)


def kernel(preds, targets):
    n_min = targets.size // 16
    kf = float(n_min)
    ntot = float(targets.size)
    scal = _run_scal(preds, targets)
    hc = scal[2, 0]

    def _hard_path(p, t, sc):
        return sc[4:5, 0:1]

    def _topk_path(p, t, sc):
        ce, bn = _run_ce(p, t)
        cnt_h, sum_h = _run_hist(ce.reshape(-1), bn.reshape(-1))
        return _run_select(cnt_h, sum_h, sc, kf, ntot)

    out = lax.cond(hc >= kf, _hard_path, _topk_path, preds, targets, scal)
    return out[0, 0]


# speedup vs baseline: 1.4309x; 1.2116x over previous
import functools

import jax
import jax.numpy as jnp
from jax import lax
from jax.experimental import pallas as pl
from jax.experimental.pallas import tpu as pltpu
from jax.experimental.pallas import tpu_sc as plsc

HB = 256
NBINS = 32768
NW = 32
BUF = 8192


def _ce_block(x, t):
    m = jnp.max(x, axis=0)
    e = jnp.exp(x - m[None, :, :])
    se = jnp.sum(e, axis=0)
    lse = m + jnp.log(se)
    cls = lax.broadcasted_iota(jnp.int32, x.shape, 0)
    xt = jnp.sum(jnp.where(cls == t[None, :, :], x, 0.0), axis=0)
    ce = jnp.maximum(lse - xt, 0.0)
    return ce, xt, m, e, se


def _scal_body(preds_ref, tgt_ref, scal_ref, *, ntot):
    b = pl.program_id(0)
    h = pl.program_id(1)
    C = preds_ref.shape[1]
    W = preds_ref.shape[3]
    m = jnp.max(preds_ref[0], axis=0)
    thresh = -jnp.log(jnp.float32(0.7))
    zero = jnp.zeros((8, W), jnp.float32)
    s_rcp = jnp.float32(0.0)
    i_part = jnp.float32(0.0)
    hc_part = jnp.float32(0.0)
    hs_part = jnp.float32(0.0)
    for r in range(HB // 8):
        rs = r * 8
        m_sub = lax.slice(m, (rs, 0), (rs + 8, W))
        a_sub = jnp.exp(jnp.minimum(m_sub, 80.0))
        t_sub = tgt_ref[0, rs:rs + 8, :]
        se = zero
        srcp = zero
        xt = zero
        for c in range(C):
            xc = preds_ref[0, c, rs:rs + 8, :]
            ec = jnp.exp(xc - m_sub)
            se = se + ec
            srcp = srcp + 1.0 / (1.0 + ec * a_sub)
            xt = xt + jnp.where(t_sub == c, xc, 0.0)
        lse = m_sub + jnp.log(se)
        ce = jnp.maximum(lse - xt, 0.0)
        hard = ce > thresh
        s_rcp = s_rcp + jnp.sum(srcp)
        i_part = i_part + jnp.sum(1.0 / (1.0 + jnp.exp(-xt)))
        hc_part = hc_part + jnp.sum(hard.astype(jnp.float32))
        hs_part = hs_part + jnp.sum(jnp.where(hard, ce, 0.0))
    s_part = jnp.float32(C * HB * W) - s_rcp

    @pl.when((b == 0) & (h == 0))
    def _init():
        scal_ref[...] = jnp.zeros_like(scal_ref)

    row = lambda v: jnp.full((1, 128), v, jnp.float32)
    upd = jnp.concatenate(
        [row(s_part), row(i_part), row(hc_part), row(hs_part),
         jnp.zeros((4, 128), jnp.float32)], axis=0)
    scal_ref[...] += upd

    @pl.when((b == pl.num_programs(0) - 1) & (h == pl.num_programs(1) - 1))
    def _final():
        iouloss, mean_hard, _ = _iou_and_hard(scal_ref, ntot)
        scal_ref[4:5, :] = jnp.full((1, 128), 2.0 * iouloss + mean_hard,
                                    jnp.float32)


def _run_scal(preds, targets):
    B, C, H, W = preds.shape
    return pl.pallas_call(
        functools.partial(_scal_body, ntot=float(targets.size)),
        grid=(B, H // HB),
        in_specs=[
            pl.BlockSpec((1, C, HB, W), lambda b, h: (b, 0, h, 0)),
            pl.BlockSpec((1, HB, W), lambda b, h: (b, h, 0)),
        ],
        out_specs=pl.BlockSpec((8, 128), lambda b, h: (0, 0)),
        out_shape=jax.ShapeDtypeStruct((8, 128), jnp.float32),
    )(preds, targets)


def _ce_body(preds_ref, tgt_ref, ce_ref, bn_ref):
    x = preds_ref[0]
    t = tgt_ref[0]
    ce, _, _, _, _ = _ce_block(x, t)
    ce_ref[0] = ce
    bn_ref[0] = lax.shift_right_logical(
        lax.bitcast_convert_type(ce, jnp.int32), 16)


def _run_ce(preds, targets):
    B, C, H, W = preds.shape
    return pl.pallas_call(
        _ce_body,
        grid=(B, H // HB),
        in_specs=[
            pl.BlockSpec((1, C, HB, W), lambda b, h: (b, 0, h, 0)),
            pl.BlockSpec((1, HB, W), lambda b, h: (b, h, 0)),
        ],
        out_specs=[
            pl.BlockSpec((1, HB, W), lambda b, h: (b, h, 0)),
            pl.BlockSpec((1, HB, W), lambda b, h: (b, h, 0)),
        ],
        out_shape=[
            jax.ShapeDtypeStruct((B, H, W), jnp.float32),
            jax.ShapeDtypeStruct((B, H, W), jnp.int32),
        ],
    )(preds, targets)


def _hist_body(ce_hbm, bn_hbm, zeros_hbm, cnt_hbm, sum_hbm,
               buf_v, bnbuf_v, hcnt_v, hsum_v):
    chunk = ce_hbm.shape[0] // NW
    c = lax.axis_index("c")
    s = lax.axis_index("s")
    wid = s * 2 + c
    base = wid * chunk
    pltpu.sync_copy(zeros_hbm, hcnt_v)
    pltpu.sync_copy(zeros_hbm, hsum_v)
    ones = jnp.full((16,), 1.0, jnp.float32)

    def outer(ib, carry):
        pltpu.sync_copy(ce_hbm.at[pl.ds(base + ib * BUF, BUF)], buf_v)
        pltpu.sync_copy(bn_hbm.at[pl.ds(base + ib * BUF, BUF)], bnbuf_v)

        def inner(j, carry2):
            for u in range(4):
                v = buf_v[pl.ds(j * 64 + u * 16, 16)]
                bn = bnbuf_v[pl.ds(j * 64 + u * 16, 16)]
                plsc.addupdate_scatter(hcnt_v, [bn], ones)
                plsc.addupdate_scatter(hsum_v, [bn], v)
            return carry2

        lax.fori_loop(0, BUF // 64, inner, 0)
        return carry

    lax.fori_loop(0, chunk // BUF, outer, 0)
    pltpu.sync_copy(hcnt_v, cnt_hbm.at[wid])
    pltpu.sync_copy(hsum_v, sum_hbm.at[wid])


def _run_hist(ce_flat, bn_flat):
    mesh = plsc.VectorSubcoreMesh(core_axis_name="c", subcore_axis_name="s")
    zeros = jnp.zeros((NBINS,), jnp.float32)
    fn = functools.partial(
        pl.kernel,
        mesh=mesh,
        compiler_params=pltpu.CompilerParams(needs_layout_passes=False),
        out_type=(
            jax.ShapeDtypeStruct((NW, NBINS), jnp.float32),
            jax.ShapeDtypeStruct((NW, NBINS), jnp.float32),
        ),
        scratch_types=[
            pltpu.VMEM((BUF,), jnp.float32),
            pltpu.VMEM((BUF,), jnp.int32),
            pltpu.VMEM((NBINS,), jnp.float32),
            pltpu.VMEM((NBINS,), jnp.float32),
        ],
    )(_hist_body)
    return fn(ce_flat, bn_flat, zeros)


def _flat_suffix(rows, w_mat, us_mat):
    rowsuf = jnp.dot(rows, w_mat, preferred_element_type=jnp.float32)
    r2 = jnp.dot(us_mat, rows, preferred_element_type=jnp.float32)
    tail = jnp.sum(r2, axis=1, keepdims=True)
    return rowsuf + tail


def _pick(onehot, vals):
    return jnp.sum(jnp.where(onehot, vals, 0.0))


def _iou_and_hard(scal_ref, ntot):
    s_all = scal_ref[0, 0]
    inter = scal_ref[1, 0]
    hc = scal_ref[2, 0]
    hs = scal_ref[3, 0]
    total = s_all + ntot
    union = total - inter
    iou = (inter + 1.0) / (union + 1.0)
    iouloss = 1.0 - iou
    mean_hard = hs / jnp.maximum(hc, 1.0)
    return iouloss, mean_hard, hc


def _select_body(cnt_ref, sum_ref, scal_ref, out_ref, *, kf, ntot):
    nr = NBINS // 128
    rows = jnp.sum(cnt_ref[...], axis=0).reshape(nr, 128)
    hrows = jnp.sum(sum_ref[...], axis=0).reshape(nr, 128)
    w_mat = (lax.broadcasted_iota(jnp.int32, (128, 128), 0)
             >= lax.broadcasted_iota(jnp.int32, (128, 128), 1)
             ).astype(jnp.float32)
    us_mat = (lax.broadcasted_iota(jnp.int32, (nr, nr), 1)
              > lax.broadcasted_iota(jnp.int32, (nr, nr), 0)
              ).astype(jnp.float32)
    f_cnt = _flat_suffix(rows, w_mat, us_mat)
    f_sum = _flat_suffix(hrows, w_mat, us_mat)
    nsel = jnp.sum((f_cnt >= kf).astype(jnp.int32))
    fid = (lax.broadcasted_iota(jnp.int32, (nr, 128), 0) * 128
           + lax.broadcasted_iota(jnp.int32, (nr, 128), 1))
    onehot = fid == (nsel - 1)
    bin_cnt = _pick(onehot, rows)
    bin_sum = _pick(onehot, hrows)
    cnt_gt = _pick(onehot, f_cnt) - bin_cnt
    sum_gt = _pick(onehot, f_sum) - bin_sum
    r = kf - cnt_gt
    topk_sum = sum_gt + r * (bin_sum / jnp.maximum(bin_cnt, 1.0))
    mean_topk = topk_sum / kf

    iouloss, mean_hard, hc = _iou_and_hard(scal_ref, ntot)
    ohem = jnp.where(hc < kf, mean_topk, mean_hard)
    out_ref[0, 0] = 2.0 * iouloss + ohem


def _run_select(cnt_h, sum_h, scal, kf, ntot):
    body = functools.partial(_select_body, kf=kf, ntot=ntot)
    return pl.pallas_call(
        body,
        in_specs=[
            pl.BlockSpec(memory_space=pltpu.VMEM),
            pl.BlockSpec(memory_space=pltpu.VMEM),
            pl.BlockSpec(memory_space=pltpu.VMEM),
        ],
        out_specs=pl.BlockSpec(memory_space=pltpu.SMEM),
        out_shape=jax.ShapeDtypeStruct((1, 1), jnp.float32),
    )(cnt_h, sum_h, scal)


def kernel(preds, targets):
    n_min = targets.size // 16
    kf = float(n_min)
    ntot = float(targets.size)
    scal = _run_scal(preds, targets)
    hc = scal[2, 0]

    def _hard_path(p, t, sc):
        return sc[4:5, 0:1]

    def _topk_path(p, t, sc):
        ce, bn = _run_ce(p, t)
        cnt_h, sum_h = _run_hist(ce.reshape(-1), bn.reshape(-1))
        return _run_select(cnt_h, sum_h, sc, kf, ntot)

    out = lax.cond(hc >= kf, _hard_path, _topk_path, preds, targets, scal)
    return out[0, 0]
